# baseline (speedup 1.0000x reference)
import functools

import jax
import jax.numpy as jnp
from jax import lax
from jax.experimental import pallas as pl
from jax.experimental.pallas import tpu as pltpu
from jax.experimental.pallas import tpu_sc as plsc

_ENCODER_SIZE = 128
_PROJ_BLOCK = 8000


def _project_body(t_ref, wt_ref, b_ref, o_ref):
    acc = jnp.dot(t_ref[...], wt_ref[...], preferred_element_type=jnp.float32)
    o_ref[...] = acc + b_ref[...]

    @pl.when(pl.program_id(0) == 0)
    def _():
        o_ref[0:1, :] = b_ref[...]


def _make_project(item_num: int, embed_dim: int):
    grid = (item_num // _PROJ_BLOCK,)
    return pl.pallas_call(
        _project_body,
        grid=grid,
        in_specs=[
            pl.BlockSpec((_PROJ_BLOCK, embed_dim), lambda i: (i, 0)),
            pl.BlockSpec((embed_dim, _ENCODER_SIZE), lambda i: (0, 0)),
            pl.BlockSpec((1, _ENCODER_SIZE), lambda i: (0, 0)),
        ],
        out_specs=pl.BlockSpec((_PROJ_BLOCK, _ENCODER_SIZE), lambda i: (i, 0)),
        out_shape=jax.ShapeDtypeStruct((item_num, _ENCODER_SIZE), jnp.float32),
    )


def _make_sc_gather(batch: int, num_docs: int):
    info = plsc.get_sparse_core_info()
    nw = info.num_cores * info.num_subcores
    bpw = batch // nw
    assert bpw * nw == batch

    mesh = plsc.VectorSubcoreMesh(core_axis_name="c", subcore_axis_name="s")

    @functools.partial(
        pl.kernel,
        out_type=jax.ShapeDtypeStruct(
            (batch, num_docs, _ENCODER_SIZE), jnp.float32
        ),
        mesh=mesh,
        scratch_types=[
            pltpu.VMEM((bpw, num_docs), jnp.int32),
            pltpu.VMEM((num_docs, _ENCODER_SIZE), jnp.float32),
            pltpu.VMEM((num_docs, _ENCODER_SIZE), jnp.float32),
            pltpu.SemaphoreType.DMA,
            pltpu.SemaphoreType.DMA,
            pltpu.SemaphoreType.DMA,
            pltpu.SemaphoreType.DMA,
        ],
        compiler_params=pltpu.CompilerParams(needs_layout_passes=False),
    )
    def sc_gather(
        ids_hbm, ptb_hbm, out_hbm, ids_v, rows0, rows1, g0, g1, o0, o1
    ):
        wid = lax.axis_index("s") * info.num_cores + lax.axis_index("c")
        base = wid * bpw
        pltpu.sync_copy(ids_hbm.at[pl.ds(base, bpw)], ids_v)

        def gather(j, buf, sem):
            pltpu.async_copy(ptb_hbm.at[ids_v.at[j]], buf, sem)

        def gather_wait(buf, sem):
            pltpu.make_async_copy(ptb_hbm.at[ids_v.at[0]], buf, sem).wait()

        def put(j, buf, sem):
            pltpu.async_copy(buf, out_hbm.at[base + j], sem)

        def put_wait(j, buf, sem):
            pltpu.make_async_copy(buf, out_hbm.at[base + j], sem).wait()

        gather(0, rows0, g0)
        gather(1, rows1, g1)

        def pair(k, _):
            j = 2 * k
            gather_wait(rows0, g0)
            put(j, rows0, o0)
            gather_wait(rows1, g1)
            put(j + 1, rows1, o1)
            put_wait(j, rows0, o0)

            @pl.when(j + 2 < bpw)
            def _():
                gather(j + 2, rows0, g0)

            put_wait(j + 1, rows1, o1)

            @pl.when(j + 3 < bpw)
            def _():
                gather(j + 3, rows1, g1)

            return ()

        lax.fori_loop(0, bpw // 2, pair, ())

    return sc_gather


def kernel(news_ids, table, W, b):
    batch, num_docs = news_ids.shape
    item_num, embed_dim = table.shape

    ptb = _make_project(item_num, embed_dim)(
        table, W.T, b.reshape(1, _ENCODER_SIZE)
    )
    return _make_sc_gather(batch, num_docs)(news_ids.astype(jnp.int32), ptb)


# speedup vs baseline: 2.0179x; 2.0179x over previous
import functools

import jax
import jax.numpy as jnp
from jax import lax
from jax.experimental import pallas as pl
from jax.experimental.pallas import tpu as pltpu
from jax.experimental.pallas import tpu_sc as plsc

_ENCODER_SIZE = 128
_PROJ_BLOCK = 4096
_CHUNK = 128


def _project_body(tt_ref, wt_ref, b_ref, o_ref):
    acc = lax.dot_general(
        tt_ref[...],
        wt_ref[...],
        dimension_numbers=(((0,), (0,)), ((), ())),
        preferred_element_type=jnp.float32,
    )
    o_ref[...] = acc + b_ref[...]

    @pl.when(pl.program_id(0) == 0)
    def _():
        o_ref[0:1, :] = b_ref[...]


def _make_project(item_num: int, embed_dim: int):
    grid = (pl.cdiv(item_num, _PROJ_BLOCK),)
    return pl.pallas_call(
        _project_body,
        grid=grid,
        in_specs=[
            pl.BlockSpec((embed_dim, _PROJ_BLOCK), lambda i: (0, i)),
            pl.BlockSpec((embed_dim, _ENCODER_SIZE), lambda i: (0, 0)),
            pl.BlockSpec((1, _ENCODER_SIZE), lambda i: (0, 0)),
        ],
        out_specs=pl.BlockSpec((_PROJ_BLOCK, _ENCODER_SIZE), lambda i: (i, 0)),
        out_shape=jax.ShapeDtypeStruct((item_num, _ENCODER_SIZE), jnp.float32),
    )


def _make_sc_gather(batch: int, num_docs: int):
    info = plsc.get_sparse_core_info()
    nw = info.num_cores * info.num_subcores
    chunks = batch // _CHUNK
    cpw = chunks // nw
    assert cpw * nw == chunks

    mesh = plsc.VectorSubcoreMesh(core_axis_name="c", subcore_axis_name="s")

    @functools.partial(
        pl.kernel,
        out_type=jax.ShapeDtypeStruct(
            (num_docs, batch, _ENCODER_SIZE), jnp.float32
        ),
        mesh=mesh,
        scratch_types=[
            pltpu.VMEM((num_docs, cpw * _CHUNK), jnp.int32),
            pltpu.VMEM((_CHUNK, _ENCODER_SIZE), jnp.float32),
            pltpu.VMEM((_CHUNK, _ENCODER_SIZE), jnp.float32),
            pltpu.SemaphoreType.DMA,
            pltpu.SemaphoreType.DMA,
            pltpu.SemaphoreType.DMA,
            pltpu.SemaphoreType.DMA,
        ],
    )
    def sc_gather(
        ids_hbm, ptb_hbm, out_hbm, ids_v, rows0, rows1, g0, g1, o0, o1
    ):
        wid = lax.axis_index("s") * info.num_cores + lax.axis_index("c")
        bbase = wid * cpw * _CHUNK
        pltpu.sync_copy(
            ids_hbm.at[:, pl.ds(bbase, cpw * _CHUNK)], ids_v
        )

        ntasks = num_docs * cpw

        def task_refs(t):
            d = t // cpw
            c = t % cpw
            idx = ids_v.at[d, pl.ds(c * _CHUNK, _CHUNK)]
            dst = out_hbm.at[d, pl.ds(bbase + c * _CHUNK, _CHUNK)]
            return idx, dst

        def gather(t, buf, sem):
            idx, _ = task_refs(t)
            pltpu.async_copy(ptb_hbm.at[idx], buf, sem)

        def gather_wait(buf, sem):
            idx, _ = task_refs(0)
            pltpu.make_async_copy(ptb_hbm.at[idx], buf, sem).wait()

        def put(t, buf, sem):
            _, dst = task_refs(t)
            pltpu.async_copy(buf, dst, sem)

        def put_wait(t, buf, sem):
            _, dst = task_refs(t)
            pltpu.make_async_copy(buf, dst, sem).wait()

        gather(0, rows0, g0)
        gather(1, rows1, g1)

        def pair(k, _):
            t = 2 * k
            gather_wait(rows0, g0)
            put(t, rows0, o0)
            gather_wait(rows1, g1)
            put(t + 1, rows1, o1)
            put_wait(t, rows0, o0)

            @pl.when(t + 2 < ntasks)
            def _():
                gather(t + 2, rows0, g0)

            put_wait(t + 1, rows1, o1)

            @pl.when(t + 3 < ntasks)
            def _():
                gather(t + 3, rows1, g1)

            return ()

        lax.fori_loop(0, ntasks // 2, pair, ())

    return sc_gather


def kernel(news_ids, table, W, b):
    batch, num_docs = news_ids.shape
    item_num, embed_dim = table.shape

    ptb = _make_project(item_num, embed_dim)(
        table.T, W.T, b.reshape(1, _ENCODER_SIZE)
    )
    out_t = _make_sc_gather(batch, num_docs)(
        news_ids.T.astype(jnp.int32), ptb
    )
    return out_t.transpose(1, 0, 2)


# speedup vs baseline: 2.3841x; 1.1815x over previous
import functools

import jax
import jax.numpy as jnp
from jax import lax
from jax.experimental import pallas as pl
from jax.experimental.pallas import tpu as pltpu
from jax.experimental.pallas import tpu_sc as plsc

_ENCODER_SIZE = 128
_PROJ_BLOCK = 8192
_CHUNK = 128


def _project_body(tt_ref, wt_ref, b_ref, o_ref):
    acc = lax.dot_general(
        tt_ref[...],
        wt_ref[...],
        dimension_numbers=(((0,), (0,)), ((), ())),
        preferred_element_type=jnp.float32,
    )
    o_ref[...] = acc + b_ref[...]

    @pl.when(pl.program_id(0) == 0)
    def _():
        o_ref[0:1, :] = b_ref[...]


def _make_project(item_num: int, embed_dim: int):
    grid = (pl.cdiv(item_num, _PROJ_BLOCK),)
    return pl.pallas_call(
        _project_body,
        grid=grid,
        in_specs=[
            pl.BlockSpec((embed_dim, _PROJ_BLOCK), lambda i: (0, i)),
            pl.BlockSpec((embed_dim, _ENCODER_SIZE), lambda i: (0, 0)),
            pl.BlockSpec((1, _ENCODER_SIZE), lambda i: (0, 0)),
        ],
        out_specs=pl.BlockSpec((_PROJ_BLOCK, _ENCODER_SIZE), lambda i: (i, 0)),
        out_shape=jax.ShapeDtypeStruct((item_num, _ENCODER_SIZE), jnp.float32),
    )


def _make_sc_gather(batch: int, num_docs: int):
    info = plsc.get_sparse_core_info()
    nw = info.num_cores * info.num_subcores
    chunks = batch // _CHUNK
    cpw = chunks // nw
    assert cpw * nw == chunks

    mesh = plsc.VectorSubcoreMesh(core_axis_name="c", subcore_axis_name="s")

    @functools.partial(
        pl.kernel,
        out_type=jax.ShapeDtypeStruct(
            (num_docs, batch, _ENCODER_SIZE), jnp.float32
        ),
        mesh=mesh,
        scratch_types=[
            pltpu.VMEM((num_docs, cpw * _CHUNK), jnp.int32),
        ]
        + [pltpu.VMEM((_CHUNK, _ENCODER_SIZE), jnp.float32)] * 4
        + [pltpu.SemaphoreType.DMA] * 8,
    )
    def sc_gather(ids_hbm, ptb_hbm, out_hbm, ids_v, *bufs_and_sems):
        rows = list(bufs_and_sems[0:4])
        g = list(bufs_and_sems[4:8])
        o = list(bufs_and_sems[8:12])
        wid = lax.axis_index("s") * info.num_cores + lax.axis_index("c")
        bbase = wid * cpw * _CHUNK
        pltpu.sync_copy(
            ids_hbm.at[:, pl.ds(bbase, cpw * _CHUNK)], ids_v
        )

        ntasks = num_docs * cpw

        def task_refs(t):
            d = t // cpw
            c = t % cpw
            idx = ids_v.at[d, pl.ds(c * _CHUNK, _CHUNK)]
            dst = out_hbm.at[d, pl.ds(bbase + c * _CHUNK, _CHUNK)]
            return idx, dst

        def gather(t, buf, sem):
            idx, _ = task_refs(t)
            pltpu.async_copy(ptb_hbm.at[idx], buf, sem)

        def gather_wait(buf, sem):
            idx, _ = task_refs(0)
            pltpu.make_async_copy(ptb_hbm.at[idx], buf, sem).wait()

        def put(t, buf, sem):
            _, dst = task_refs(t)
            pltpu.async_copy(buf, dst, sem)

        def put_wait(t, buf, sem):
            _, dst = task_refs(t)
            pltpu.make_async_copy(buf, dst, sem).wait()

        for b in range(4):
            gather(b, rows[b], g[b])

        def quad(k, _):
            t0 = 4 * k
            for b in range(4):
                gather_wait(rows[b], g[b])
                put(t0 + b, rows[b], o[b])
            for b in range(4):
                put_wait(t0 + b, rows[b], o[b])

                @pl.when(t0 + b + 4 < ntasks)
                def _(b=b):
                    gather(t0 + b + 4, rows[b], g[b])

            return ()

        lax.fori_loop(0, ntasks // 4, quad, ())

    return sc_gather


def kernel(news_ids, table, W, b):
    batch, num_docs = news_ids.shape
    item_num, embed_dim = table.shape

    ptb = _make_project(item_num, embed_dim)(
        table.T, W.T, b.reshape(1, _ENCODER_SIZE)
    )
    out_t = _make_sc_gather(batch, num_docs)(
        news_ids.T.astype(jnp.int32), ptb
    )
    return out_t.transpose(1, 0, 2)


# speedup vs baseline: 2.4875x; 1.0434x over previous
import functools

import jax
import jax.numpy as jnp
from jax import lax
from jax.experimental import pallas as pl
from jax.experimental.pallas import tpu as pltpu
from jax.experimental.pallas import tpu_sc as plsc

_ENCODER_SIZE = 128
_PROJ_BLOCK = 16384
_CHUNK = 128


def _project_body(tt_ref, wt_ref, b_ref, o_ref):
    acc = lax.dot_general(
        tt_ref[...],
        wt_ref[...],
        dimension_numbers=(((0,), (0,)), ((), ())),
        preferred_element_type=jnp.float32,
    )
    o_ref[...] = acc + b_ref[...]

    @pl.when(pl.program_id(0) == 0)
    def _():
        o_ref[0:1, :] = b_ref[...]


def _make_project(item_num: int, embed_dim: int):
    grid = (pl.cdiv(item_num, _PROJ_BLOCK),)
    return pl.pallas_call(
        _project_body,
        grid=grid,
        in_specs=[
            pl.BlockSpec((embed_dim, _PROJ_BLOCK), lambda i: (0, i)),
            pl.BlockSpec((embed_dim, _ENCODER_SIZE), lambda i: (0, 0)),
            pl.BlockSpec((1, _ENCODER_SIZE), lambda i: (0, 0)),
        ],
        out_specs=pl.BlockSpec((_PROJ_BLOCK, _ENCODER_SIZE), lambda i: (i, 0)),
        out_shape=jax.ShapeDtypeStruct((item_num, _ENCODER_SIZE), jnp.float32),
    )


def _make_sc_gather(batch: int, num_docs: int):
    info = plsc.get_sparse_core_info()
    nw = info.num_cores * info.num_subcores
    chunks = batch // _CHUNK
    cpw = chunks // nw
    assert cpw * nw == chunks

    mesh = plsc.VectorSubcoreMesh(core_axis_name="c", subcore_axis_name="s")

    @functools.partial(
        pl.kernel,
        out_type=jax.ShapeDtypeStruct(
            (num_docs, batch, _ENCODER_SIZE), jnp.float32
        ),
        mesh=mesh,
        scratch_types=[
            pltpu.VMEM((num_docs, cpw * _CHUNK), jnp.int32),
        ]
        + [pltpu.VMEM((_CHUNK, _ENCODER_SIZE), jnp.float32)] * 4
        + [pltpu.SemaphoreType.DMA] * 8,
    )
    def sc_gather(ids_hbm, ptb_hbm, out_hbm, ids_v, *bufs_and_sems):
        rows = list(bufs_and_sems[0:4])
        g = list(bufs_and_sems[4:8])
        o = list(bufs_and_sems[8:12])
        wid = lax.axis_index("s") * info.num_cores + lax.axis_index("c")
        bbase = wid * cpw * _CHUNK
        pltpu.sync_copy(
            ids_hbm.at[:, pl.ds(bbase, cpw * _CHUNK)], ids_v
        )

        ntasks = num_docs * cpw

        def task_refs(t):
            d = t // cpw
            c = t % cpw
            idx = ids_v.at[d, pl.ds(c * _CHUNK, _CHUNK)]
            dst = out_hbm.at[d, pl.ds(bbase + c * _CHUNK, _CHUNK)]
            return idx, dst

        def gather(t, buf, sem):
            idx, _ = task_refs(t)
            pltpu.async_copy(ptb_hbm.at[idx], buf, sem)

        def gather_wait(buf, sem):
            idx, _ = task_refs(0)
            pltpu.make_async_copy(ptb_hbm.at[idx], buf, sem).wait()

        def put(t, buf, sem):
            _, dst = task_refs(t)
            pltpu.async_copy(buf, dst, sem)

        def put_wait(t, buf, sem):
            _, dst = task_refs(t)
            pltpu.make_async_copy(buf, dst, sem).wait()

        for b in range(4):
            gather(b, rows[b], g[b])

        def quad(k, _):
            t0 = 4 * k
            for b in range(4):
                gather_wait(rows[b], g[b])
                put(t0 + b, rows[b], o[b])
            for b in range(4):
                put_wait(t0 + b, rows[b], o[b])

                @pl.when(t0 + b + 4 < ntasks)
                def _(b=b):
                    gather(t0 + b + 4, rows[b], g[b])

            return ()

        lax.fori_loop(0, ntasks // 4, quad, ())

    return sc_gather


def kernel(news_ids, table, W, b):
    batch, num_docs = news_ids.shape
    item_num, embed_dim = table.shape

    ptb = _make_project(item_num, embed_dim)(
        table.T, W.T, b.reshape(1, _ENCODER_SIZE)
    )
    out_t = _make_sc_gather(batch, num_docs)(
        news_ids.T.astype(jnp.int32), ptb
    )
    return out_t.transpose(1, 0, 2)
